# baseline (speedup 1.0000x reference)
import functools

import jax
import jax.numpy as jnp
from jax import lax
from jax.experimental import pallas as pl
from jax.experimental.pallas import tpu as pltpu
from jax.experimental.pallas import tpu_sc as plsc

_NW = 32
_LANES = 16


def _sc_d2(xs, ys, zs, src, dst):
    E = src.shape[0]
    N_NODES = xs.shape[0]
    per_w = E // _NW
    C = 2000
    assert per_w % C == 0 and C % _LANES == 0
    n_chunks = per_w // C
    assert n_chunks % 2 == 1 and n_chunks >= 3
    groups = C // _LANES

    mesh = plsc.VectorSubcoreMesh(core_axis_name="c", subcore_axis_name="s")

    @functools.partial(
        pl.kernel,
        mesh=mesh,
        out_type=jax.ShapeDtypeStruct((E,), jnp.float32),
        scratch_types=[
            pltpu.VMEM((2 * C,), jnp.int32),
            pltpu.VMEM((2 * C,), jnp.int32),
            pltpu.VMEM((2 * C,), jnp.float32),
            pltpu.VMEM((2 * C,), jnp.float32),
            pltpu.VMEM((2 * C,), jnp.float32),
            pltpu.VMEM((2 * C,), jnp.float32),
            pltpu.VMEM((2 * C,), jnp.float32),
            pltpu.VMEM((2 * C,), jnp.float32),
            pltpu.VMEM((C,), jnp.float32),
            pltpu.VMEM((C,), jnp.float32),
            pltpu.VMEM_SHARED((N_NODES,), jnp.float32),
            pltpu.VMEM_SHARED((N_NODES,), jnp.float32),
            pltpu.VMEM_SHARED((N_NODES,), jnp.float32),
            pltpu.SemaphoreType.DMA,
            pltpu.SemaphoreType.DMA,
            pltpu.SemaphoreType.DMA,
            pltpu.SemaphoreType.DMA,
            pltpu.SemaphoreType.DMA,
            pltpu.SemaphoreType.DMA,
        ],
    )
    def sc_kernel(xs_hbm, ys_hbm, zs_hbm, src_hbm, dst_hbm, d2_hbm,
                  idx0, idx1, xb0, yb0, zb0, xb1, yb1, zb1, d20, d21,
                  xs_sh, ys_sh, zs_sh,
                  semi0, semi1, semg0, semg1, semd0, semd1):
        wid = lax.axis_index("s") * 2 + lax.axis_index("c")
        base = wid * per_w
        @pl.when(lax.axis_index("s") == 0)
        def _():
            pltpu.sync_copy(xs_hbm, xs_sh)
            pltpu.sync_copy(ys_hbm, ys_sh)
            pltpu.sync_copy(zs_hbm, zs_sh)
        plsc.subcore_barrier()
        idx = (idx0, idx1)
        xb = (xb0, xb1)
        yb = (yb0, yb1)
        zb = (zb0, zb1)
        d2c = (d20, d21)
        semi = (semi0, semi1)
        semg = (semg0, semg1)
        semd = (semd0, semd1)
        planes = (xs_sh, ys_sh, zs_sh)

        def fire_idx(ci, b):
            cb = base + ci * C
            pltpu.async_copy(src_hbm.at[pl.ds(cb, C)], idx[b].at[pl.ds(0, C)],
                             semi[b])
            pltpu.async_copy(dst_hbm.at[pl.ds(cb, C)], idx[b].at[pl.ds(C, C)],
                             semi[b])

        def wait_idx(b):
            cb0 = base
            pltpu.make_async_copy(src_hbm.at[pl.ds(cb0, C)],
                                  idx[b].at[pl.ds(0, C)], semi[b]).wait()
            pltpu.make_async_copy(dst_hbm.at[pl.ds(cb0, C)],
                                  idx[b].at[pl.ds(C, C)], semi[b]).wait()

        def fire_gathers(b):
            for plane, buf in zip(planes, (xb[b], yb[b], zb[b])):
                for h in (0, 1):
                    pltpu.async_copy(plane.at[idx[b].at[pl.ds(h * C, C)]],
                                     buf.at[pl.ds(h * C, C)], semg[b])

        def wait_gathers(b):
            for plane, buf in zip(planes, (xb[b], yb[b], zb[b])):
                for h in (0, 1):
                    pltpu.make_async_copy(plane.at[idx[b].at[pl.ds(h * C, C)]],
                                          buf.at[pl.ds(h * C, C)],
                                          semg[b]).wait()

        def fire_store(ci, b):
            cb = base + ci * C
            pltpu.async_copy(d2c[b], d2_hbm.at[pl.ds(cb, C)], semd[b])

        def wait_store(ci, b):
            cb = base + ci * C
            pltpu.make_async_copy(d2c[b], d2_hbm.at[pl.ds(cb, C)],
                                  semd[b]).wait()

        def compute(b):
            xbb, ybb, zbb, out = xb[b], yb[b], zb[b], d2c[b]

            def grp(gi, _):
                o = gi * _LANES
                sl = pl.ds(o, _LANES)
                sl2 = pl.ds(C + o, _LANES)
                ddx = xbb[sl] - xbb[sl2]
                ddy = ybb[sl] - ybb[sl2]
                ddz = zbb[sl] - zbb[sl2]
                out[sl] = ddx * ddx + ddy * ddy + ddz * ddz
                return 0

            lax.fori_loop(0, groups, grp, 0)

        fire_idx(0, 0)
        wait_idx(0)
        fire_gathers(0)
        fire_idx(1, 1)

        def pair_body(p, _):
            for off in (0, 1):
                b = off
                ci = 2 * p + off
                wait_gathers(b)
                @pl.when(ci + 2 < n_chunks)
                def _():
                    fire_idx(ci + 2, b)
                @pl.when(ci + 1 < n_chunks)
                def _():
                    wait_idx(1 - b)
                    fire_gathers(1 - b)
                @pl.when(ci >= 2)
                def _():
                    wait_store(ci - 2, b)
                compute(b)
                fire_store(ci, b)
            return 0

        lax.fori_loop(0, (n_chunks - 1) // 2, pair_body, 0)

        ci = n_chunks - 1
        wait_gathers(0)
        wait_store(ci - 2, 0)
        compute(0)
        fire_store(ci, 0)
        wait_store(ci - 1, 1)
        wait_store(ci, 0)

    return sc_kernel(xs, ys, zs, src, dst)


def _tc_expand(d2, shift_col, scale_col):
    E = d2.shape[0]
    NG = shift_col.shape[0]
    BL = 32768

    def body(d2_ref, sh_ref, sc_ref, out_ref):
        norm = jnp.sqrt(d2_ref[...])
        sh = sh_ref[...]
        sc = sc_ref[...]
        inv = -0.5 / (sc * sc)
        diff = norm[None, :] - sh
        out_ref[...] = jnp.exp(diff * diff * inv)

    return pl.pallas_call(
        body,
        grid=(pl.cdiv(E, BL),),
        in_specs=[
            pl.BlockSpec((BL,), lambda i: (i,)),
            pl.BlockSpec((NG, 1), lambda i: (0, 0)),
            pl.BlockSpec((NG, 1), lambda i: (0, 0)),
        ],
        out_specs=pl.BlockSpec((NG, BL), lambda i: (0, i)),
        out_shape=jax.ShapeDtypeStruct((NG, E), jnp.float32),
    )(d2, shift_col, scale_col)


def kernel(pos, edge_index, shift, scale):
    NG = shift.shape[0]
    src = edge_index[0].astype(jnp.int32)
    dst = edge_index[1].astype(jnp.int32)
    posf = pos.astype(jnp.float32)
    xs = posf[:, 0]
    ys = posf[:, 1]
    zs = posf[:, 2]
    d2 = _sc_d2(xs, ys, zs, src, dst)
    out_t = _tc_expand(d2, shift.reshape(NG, 1), scale.reshape(NG, 1))
    return out_t.T


# speedup vs baseline: 1.1769x; 1.1769x over previous
import functools

import jax
import jax.numpy as jnp
from jax import lax
from jax.experimental import pallas as pl
from jax.experimental.pallas import tpu as pltpu
from jax.experimental.pallas import tpu_sc as plsc

_NW = 32
_LANES = 16


def _sc_d2(xs, ys, zs, ei_flat, E):
    N_NODES = xs.shape[0]
    per_w = E // _NW
    C = 2000
    assert per_w % C == 0 and C % _LANES == 0
    n_chunks = per_w // C
    assert n_chunks % 2 == 1 and n_chunks >= 3
    groups = C // _LANES

    mesh = plsc.VectorSubcoreMesh(core_axis_name="c", subcore_axis_name="s")

    @functools.partial(
        pl.kernel,
        mesh=mesh,
        out_type=jax.ShapeDtypeStruct((E,), jnp.float32),
        scratch_types=[
            pltpu.VMEM((2 * C,), jnp.int32),
            pltpu.VMEM((2 * C,), jnp.int32),
            pltpu.VMEM((2 * C,), jnp.float32),
            pltpu.VMEM((2 * C,), jnp.float32),
            pltpu.VMEM((2 * C,), jnp.float32),
            pltpu.VMEM((2 * C,), jnp.float32),
            pltpu.VMEM((2 * C,), jnp.float32),
            pltpu.VMEM((2 * C,), jnp.float32),
            pltpu.VMEM((C,), jnp.float32),
            pltpu.VMEM((C,), jnp.float32),
            pltpu.VMEM_SHARED((N_NODES,), jnp.float32),
            pltpu.VMEM_SHARED((N_NODES,), jnp.float32),
            pltpu.VMEM_SHARED((N_NODES,), jnp.float32),
            pltpu.SemaphoreType.DMA,
            pltpu.SemaphoreType.DMA,
            pltpu.SemaphoreType.DMA,
            pltpu.SemaphoreType.DMA,
            pltpu.SemaphoreType.DMA,
            pltpu.SemaphoreType.DMA,
        ],
    )
    def sc_kernel(xs_hbm, ys_hbm, zs_hbm, ei_hbm, d2_hbm,
                  idx0, idx1, xb0, yb0, zb0, xb1, yb1, zb1, d20, d21,
                  xs_sh, ys_sh, zs_sh,
                  semi0, semi1, semg0, semg1, semd0, semd1):
        wid = lax.axis_index("s") * 2 + lax.axis_index("c")
        base = wid * per_w
        @pl.when(lax.axis_index("s") == 0)
        def _():
            pltpu.sync_copy(xs_hbm, xs_sh)
            pltpu.sync_copy(ys_hbm, ys_sh)
            pltpu.sync_copy(zs_hbm, zs_sh)
        plsc.subcore_barrier()
        idx = (idx0, idx1)
        xb = (xb0, xb1)
        yb = (yb0, yb1)
        zb = (zb0, zb1)
        d2c = (d20, d21)
        semi = (semi0, semi1)
        semg = (semg0, semg1)
        semd = (semd0, semd1)
        planes = (xs_sh, ys_sh, zs_sh)

        def fire_idx(ci, b):
            cb = base + ci * C
            pltpu.async_copy(ei_hbm.at[pl.ds(cb, C)], idx[b].at[pl.ds(0, C)],
                             semi[b])
            pltpu.async_copy(ei_hbm.at[pl.ds(E + cb, C)], idx[b].at[pl.ds(C, C)],
                             semi[b])

        def wait_idx(b):
            cb0 = base
            pltpu.make_async_copy(ei_hbm.at[pl.ds(cb0, C)],
                                  idx[b].at[pl.ds(0, C)], semi[b]).wait()
            pltpu.make_async_copy(ei_hbm.at[pl.ds(E + cb0, C)],
                                  idx[b].at[pl.ds(C, C)], semi[b]).wait()

        def fire_gathers(b):
            for plane, buf in zip(planes, (xb[b], yb[b], zb[b])):
                for h in (0, 1):
                    pltpu.async_copy(plane.at[idx[b].at[pl.ds(h * C, C)]],
                                     buf.at[pl.ds(h * C, C)], semg[b])

        def wait_gathers(b):
            for plane, buf in zip(planes, (xb[b], yb[b], zb[b])):
                for h in (0, 1):
                    pltpu.make_async_copy(plane.at[idx[b].at[pl.ds(h * C, C)]],
                                          buf.at[pl.ds(h * C, C)],
                                          semg[b]).wait()

        def fire_store(ci, b):
            cb = base + ci * C
            pltpu.async_copy(d2c[b], d2_hbm.at[pl.ds(cb, C)], semd[b])

        def wait_store(ci, b):
            cb = base + ci * C
            pltpu.make_async_copy(d2c[b], d2_hbm.at[pl.ds(cb, C)],
                                  semd[b]).wait()

        def compute(b):
            xbb, ybb, zbb, out = xb[b], yb[b], zb[b], d2c[b]

            def grp(gi, _):
                o = gi * _LANES
                sl = pl.ds(o, _LANES)
                sl2 = pl.ds(C + o, _LANES)
                ddx = xbb[sl] - xbb[sl2]
                ddy = ybb[sl] - ybb[sl2]
                ddz = zbb[sl] - zbb[sl2]
                out[sl] = ddx * ddx + ddy * ddy + ddz * ddz
                return 0

            lax.fori_loop(0, groups, grp, 0)

        fire_idx(0, 0)
        wait_idx(0)
        fire_gathers(0)
        fire_idx(1, 1)

        def pair_body(p, _):
            for off in (0, 1):
                b = off
                ci = 2 * p + off
                wait_gathers(b)
                @pl.when(ci + 2 < n_chunks)
                def _():
                    fire_idx(ci + 2, b)
                @pl.when(ci + 1 < n_chunks)
                def _():
                    wait_idx(1 - b)
                    fire_gathers(1 - b)
                @pl.when(ci >= 2)
                def _():
                    wait_store(ci - 2, b)
                compute(b)
                fire_store(ci, b)
            return 0

        lax.fori_loop(0, (n_chunks - 1) // 2, pair_body, 0)

        ci = n_chunks - 1
        wait_gathers(0)
        wait_store(ci - 2, 0)
        compute(0)
        fire_store(ci, 0)
        wait_store(ci - 1, 1)
        wait_store(ci, 0)

    return sc_kernel(xs, ys, zs, ei_flat)


def _tc_expand(d2, shift_col, scale_col):
    E = d2.shape[0]
    NG = shift_col.shape[0]
    BL = 32768

    def body(d2_ref, sh_ref, sc_ref, out_ref):
        norm = jnp.sqrt(d2_ref[...])
        sh = sh_ref[...]
        sc = sc_ref[...]
        inv = -0.5 / (sc * sc)
        diff = norm[None, :] - sh
        out_ref[...] = jnp.exp(diff * diff * inv)

    return pl.pallas_call(
        body,
        grid=(pl.cdiv(E, BL),),
        in_specs=[
            pl.BlockSpec((BL,), lambda i: (i,)),
            pl.BlockSpec((NG, 1), lambda i: (0, 0)),
            pl.BlockSpec((NG, 1), lambda i: (0, 0)),
        ],
        out_specs=pl.BlockSpec((NG, BL), lambda i: (0, i)),
        out_shape=jax.ShapeDtypeStruct((NG, E), jnp.float32),
    )(d2, shift_col, scale_col)


def kernel(pos, edge_index, shift, scale):
    NG = shift.shape[0]
    E = edge_index.shape[1]
    ei_flat = edge_index.astype(jnp.int32).reshape(2 * E)
    posf = pos.astype(jnp.float32)
    xs = posf[:, 0]
    ys = posf[:, 1]
    zs = posf[:, 2]
    d2 = _sc_d2(xs, ys, zs, ei_flat, E)
    out_t = _tc_expand(d2, shift.reshape(NG, 1), scale.reshape(NG, 1))
    return out_t.T
